# baseline (speedup 1.0000x reference)
import functools
import math

import jax
import jax.numpy as jnp
from jax.experimental import pallas as pl

_NUM_UNITS = 1024
_SCALE = math.sqrt(_NUM_UNITS)
_LN10000 = math.log(10000.0)
_TB = 128
_G = 8


def _pe_kernel(out_ref, *, n_batch):
    t_blk = pl.program_id(0)
    base = t_blk * _TB

    col_i = jax.lax.broadcasted_iota(jnp.int32, (_G, _NUM_UNITS), 1)
    col = col_i.astype(jnp.float32)
    inv_freq = jnp.exp(col * (-2.0 * _LN10000 / _NUM_UNITS))
    phase = (col_i & 1).astype(jnp.float32) * (0.5 * math.pi)

    c8 = jnp.cos(inv_freq * float(_G))
    s8 = jnp.sin(inv_freq * float(_G))

    row = jax.lax.broadcasted_iota(jnp.int32, (_G, _NUM_UNITS), 0) + base
    ang = row.astype(jnp.float32) * inv_freq + phase
    v = jnp.sin(ang) * _SCALE
    w = jnp.cos(ang) * _SCALE

    first = jnp.where(row == 0, 0.0, v)
    out_ref[:, 0:_G, :] = jnp.broadcast_to(first[None], (n_batch, _G, _NUM_UNITS))

    for k in range(1, _TB // _G):
        v, w = v * c8 + w * s8, w * c8 - v * s8
        out_ref[:, k * _G:(k + 1) * _G, :] = jnp.broadcast_to(
            v[None], (n_batch, _G, _NUM_UNITS))


def kernel(inputs, y):
    n, t = inputs.shape
    del y
    grid = (t // _TB,)
    out = pl.pallas_call(
        functools.partial(_pe_kernel, n_batch=n),
        grid=grid,
        out_specs=pl.BlockSpec((n, _TB, _NUM_UNITS), lambda tb: (0, tb, 0)),
        out_shape=jax.ShapeDtypeStruct((n, t, _NUM_UNITS), jnp.float32),
    )()
    return out


# speedup vs baseline: 1.2097x; 1.2097x over previous
import functools
import math

import jax
import jax.numpy as jnp
from jax.experimental import pallas as pl

_NUM_UNITS = 1024
_SCALE = math.sqrt(_NUM_UNITS)
_LN10000 = math.log(10000.0)
_TB = 1024
_G = 8


def _pe_kernel(out_ref, *, n_batch):
    t_blk = pl.program_id(0)
    base = t_blk * _TB

    col_i = jax.lax.broadcasted_iota(jnp.int32, (_G, _NUM_UNITS), 1)
    col = col_i.astype(jnp.float32)
    inv_freq = jnp.exp(col * (-2.0 * _LN10000 / _NUM_UNITS))
    phase = (col_i & 1).astype(jnp.float32) * (0.5 * math.pi)

    c8 = jnp.cos(inv_freq * float(_G))
    s8 = jnp.sin(inv_freq * float(_G))

    row = jax.lax.broadcasted_iota(jnp.int32, (_G, _NUM_UNITS), 0) + base
    ang = row.astype(jnp.float32) * inv_freq + phase
    v = jnp.sin(ang) * _SCALE
    w = jnp.cos(ang) * _SCALE

    first = jnp.where(row == 0, 0.0, v)
    out_ref[:, 0:_G, :] = jnp.broadcast_to(first[None], (n_batch, _G, _NUM_UNITS))

    for k in range(1, _TB // _G):
        v, w = v * c8 + w * s8, w * c8 - v * s8
        out_ref[:, k * _G:(k + 1) * _G, :] = jnp.broadcast_to(
            v[None], (n_batch, _G, _NUM_UNITS))


def kernel(inputs, y):
    n, t = inputs.shape
    del y
    grid = (t // _TB,)
    out = pl.pallas_call(
        functools.partial(_pe_kernel, n_batch=n),
        grid=grid,
        out_specs=pl.BlockSpec((n, _TB, _NUM_UNITS), lambda tb: (0, tb, 0)),
        out_shape=jax.ShapeDtypeStruct((n, t, _NUM_UNITS), jnp.float32),
    )()
    return out


# speedup vs baseline: 1.3077x; 1.0810x over previous
import functools
import math

import jax
import jax.numpy as jnp
from jax.experimental import pallas as pl

_NUM_UNITS = 1024
_SCALE = math.sqrt(_NUM_UNITS)
_LN10000 = math.log(10000.0)
_TB = 256
_G = 8


def _pe_kernel(out_ref, *, n_batch):
    t_blk = pl.program_id(0)
    base = t_blk * _TB

    col_i = jax.lax.broadcasted_iota(jnp.int32, (_G, _NUM_UNITS), 1)
    col = col_i.astype(jnp.float32)
    inv_freq = jnp.exp(col * (-2.0 * _LN10000 / _NUM_UNITS))
    phase = (col_i & 1).astype(jnp.float32) * (0.5 * math.pi)

    c8 = jnp.cos(inv_freq * float(_G))
    s8 = jnp.sin(inv_freq * float(_G))

    row = jax.lax.broadcasted_iota(jnp.int32, (_G, _NUM_UNITS), 0) + base
    ang = row.astype(jnp.float32) * inv_freq + phase
    v = jnp.sin(ang) * _SCALE
    w = jnp.cos(ang) * _SCALE

    first = jnp.where(row == 0, 0.0, v)
    out_ref[:, 0:_G, :] = jnp.broadcast_to(first[None], (n_batch, _G, _NUM_UNITS))

    for k in range(1, _TB // _G):
        v, w = v * c8 + w * s8, w * c8 - v * s8
        out_ref[:, k * _G:(k + 1) * _G, :] = jnp.broadcast_to(
            v[None], (n_batch, _G, _NUM_UNITS))


def kernel(inputs, y):
    n, t = inputs.shape
    del y
    grid = (t // _TB,)
    out = pl.pallas_call(
        functools.partial(_pe_kernel, n_batch=n),
        grid=grid,
        out_specs=pl.BlockSpec((n, _TB, _NUM_UNITS), lambda tb: (0, tb, 0)),
        out_shape=jax.ShapeDtypeStruct((n, t, _NUM_UNITS), jnp.float32),
    )()
    return out


# speedup vs baseline: 1.3319x; 1.0185x over previous
import functools
import math

import jax
import jax.numpy as jnp
from jax.experimental import pallas as pl
from jax.experimental.pallas import tpu as pltpu

_NUM_UNITS = 1024
_SCALE = math.sqrt(_NUM_UNITS)
_LN10000 = math.log(10000.0)
_TB = 256
_G = 8


def _pe_kernel(out_ref, v_ref, w_ref, c_ref, s_ref, *, n_batch):
    t_blk = pl.program_id(0)

    @pl.when(t_blk == 0)
    def _seed():
        col_i = jax.lax.broadcasted_iota(jnp.int32, (_G, _NUM_UNITS), 1)
        col = col_i.astype(jnp.float32)
        inv_freq = jnp.exp(col * (-2.0 * _LN10000 / _NUM_UNITS))
        phase = (col_i & 1).astype(jnp.float32) * (0.5 * math.pi)
        c_ref[...] = jnp.cos(inv_freq * float(_G))
        s_ref[...] = jnp.sin(inv_freq * float(_G))
        row = jax.lax.broadcasted_iota(jnp.int32, (_G, _NUM_UNITS), 0)
        ang = row.astype(jnp.float32) * inv_freq + phase
        v_ref[...] = jnp.sin(ang) * _SCALE
        w_ref[...] = jnp.cos(ang) * _SCALE

    v = v_ref[...]
    w = w_ref[...]
    c8 = c_ref[...]
    s8 = s_ref[...]

    row = jax.lax.broadcasted_iota(jnp.int32, (_G, _NUM_UNITS), 0) + t_blk * _TB
    first = jnp.where(row == 0, 0.0, v)
    out_ref[:, 0:_G, :] = jnp.broadcast_to(first[None], (n_batch, _G, _NUM_UNITS))

    for k in range(1, _TB // _G):
        v, w = v * c8 + w * s8, w * c8 - v * s8
        out_ref[:, k * _G:(k + 1) * _G, :] = jnp.broadcast_to(
            v[None], (n_batch, _G, _NUM_UNITS))

    v_ref[...], w_ref[...] = v * c8 + w * s8, w * c8 - v * s8


def kernel(inputs, y):
    n, t = inputs.shape
    del y
    grid = (t // _TB,)
    out = pl.pallas_call(
        functools.partial(_pe_kernel, n_batch=n),
        grid=grid,
        out_specs=pl.BlockSpec((n, _TB, _NUM_UNITS), lambda tb: (0, tb, 0)),
        out_shape=jax.ShapeDtypeStruct((n, t, _NUM_UNITS), jnp.float32),
        scratch_shapes=[pltpu.VMEM((_G, _NUM_UNITS), jnp.float32)] * 4,
        compiler_params=pltpu.CompilerParams(
            dimension_semantics=("arbitrary",)),
    )()
    return out
